# baseline (speedup 1.0000x reference)
import functools

import jax
import jax.numpy as jnp
from jax import lax
from jax.experimental import pallas as pl
from jax.experimental.pallas import tpu as pltpu
from jax.experimental.pallas import tpu_sc as plsc

B, T, F, H, FS, OD, TK = 1024, 50, 8, 64, 8, 1, 4
M = 100000
MT = 2048
M_PAD = ((M + MT - 1) // MT) * MT
N_TILES = M_PAD // MT
BIG_I = 2**30
INF = float("inf")


def _lex_lt(v1, i1, v2, i2):
    return (v1 < v2) | ((v1 == v2) & (i1 < i2))


def _topk_kernel(q_ref, metaT_ref, idx_ref, bv_ref, bi_ref):
    pid = pl.program_id(0)

    @pl.when(pid == 0)
    def _init():
        bv_ref[...] = jnp.full((B, 8), INF, jnp.float32)
        bi_ref[...] = jnp.full((B, 8), BIG_I, jnp.int32)

    mt = metaT_ref[...]
    norms = jnp.sum(mt * mt, axis=0, keepdims=True)
    qm = jnp.dot(q_ref[...], mt, preferred_element_type=jnp.float32)
    s = norms - 2.0 * qm

    lane = lax.broadcasted_iota(jnp.int32, (1, MT), 1)
    off = pid * MT
    gidx = lane + off
    s = jnp.where(gidx < M, s, INF)

    new_v, new_i = [], []
    for _ in range(TK):
        m = jnp.min(s, axis=1, keepdims=True)
        am = jnp.min(jnp.where(s == m, lane, BIG_I), axis=1, keepdims=True)
        new_v.append(m)
        new_i.append(am + off)
        s = jnp.where(lane == am, INF, s)

    cand_v = [bv_ref[:, k][:, None] for k in range(TK)] + new_v
    cand_i = [bi_ref[:, k][:, None] for k in range(TK)] + new_i
    out_v, out_i = [], []
    for _ in range(TK):
        best_v, best_i = cand_v[0], cand_i[0]
        for j in range(1, 2 * TK):
            take = _lex_lt(cand_v[j], cand_i[j], best_v, best_i)
            best_v = jnp.where(take, cand_v[j], best_v)
            best_i = jnp.where(take, cand_i[j], best_i)
        out_v.append(best_v)
        out_i.append(best_i)
        for j in range(2 * TK):
            hit = cand_i[j] == best_i
            cand_v[j] = jnp.where(hit, INF, cand_v[j])
            cand_i[j] = jnp.where(hit, BIG_I, cand_i[j])
    pad_v = [jnp.full((B, 1), INF, jnp.float32)] * (8 - TK)
    pad_i = [jnp.full((B, 1), BIG_I, jnp.int32)] * (8 - TK)
    bv_ref[...] = jnp.concatenate(out_v + pad_v, axis=1)
    bi_ref[...] = jnp.concatenate(out_i + pad_i, axis=1)

    @pl.when(pid == N_TILES - 1)
    def _fin():
        idx_ref[...] = bi_ref[...]


def _topk(q, metaT_pad):
    return pl.pallas_call(
        _topk_kernel,
        grid=(N_TILES,),
        in_specs=[
            pl.BlockSpec((B, T), lambda i: (0, 0)),
            pl.BlockSpec((T, MT), lambda i: (0, i)),
        ],
        out_specs=pl.BlockSpec((B, 8), lambda i: (0, 0)),
        out_shape=jax.ShapeDtypeStruct((B, 8), jnp.int32),
        scratch_shapes=[
            pltpu.VMEM((B, 8), jnp.float32),
            pltpu.VMEM((B, 8), jnp.int32),
        ],
    )(q, metaT_pad)


_SC_INFO = plsc.get_sparse_core_info()
_NC, _NS = _SC_INFO.num_cores, _SC_INFO.num_subcores
_NW = _NC * _NS
_BTK = B * TK
_BPW = _BTK // _NW
_DPAD = 128


def _sc_gather(table_pad, idx_flat):
    mesh = plsc.VectorSubcoreMesh(core_axis_name="c", subcore_axis_name="s")

    @functools.partial(
        pl.kernel, mesh=mesh,
        out_type=jax.ShapeDtypeStruct((_BTK, _DPAD), jnp.float32),
        scratch_types=[
            pltpu.VMEM((_BPW,), jnp.int32),
            pltpu.VMEM((_BPW, _DPAD), jnp.float32),
            pltpu.SemaphoreType.DMA,
        ],
    )
    def k(table_hbm, idx_hbm, out_hbm, idx_v, rows_v, sem):
        wid = lax.axis_index("s") * _NC + lax.axis_index("c")
        base = wid * _BPW
        pltpu.sync_copy(idx_hbm.at[pl.ds(base, _BPW)], idx_v)
        pltpu.async_copy(table_hbm.at[idx_v], rows_v, sem).wait()
        pltpu.sync_copy(rows_v, out_hbm.at[pl.ds(base, _BPW)])

    return k(table_pad, idx_flat)


def _forecast_kernel(xT_ref, q_ref, rag_ref,
                     WihT_ref, WhhT_ref, bih_ref, bhh_ref,
                     foWT_ref, fob_ref, rfWT_ref, rfb_ref,
                     gW1T_ref, gb1_ref, gW2T_ref, gb2_ref,
                     out_ref):
    WihT = WihT_ref[...]
    WhhT = WhhT_ref[...]
    bih = bih_ref[...]
    bhh = bhh_ref[...]

    def step(t, h):
        xt = xT_ref[t]
        gi = jnp.dot(xt, WihT, preferred_element_type=jnp.float32) + bih
        gh = jnp.dot(h, WhhT, preferred_element_type=jnp.float32) + bhh
        r = jax.nn.sigmoid(gi[:, :H] + gh[:, :H])
        z = jax.nn.sigmoid(gi[:, H:2 * H] + gh[:, H:2 * H])
        n = jnp.tanh(gi[:, 2 * H:] + r * gh[:, 2 * H:])
        return (1.0 - z) * n + z * h

    h = lax.fori_loop(0, T, step, jnp.zeros((B, H), jnp.float32))

    q = q_ref[...]
    t1 = jnp.tanh(jnp.dot(q, gW1T_ref[...], preferred_element_type=jnp.float32) + gb1_ref[...])
    gate = jax.nn.sigmoid(jnp.dot(t1, gW2T_ref[...], preferred_element_type=jnp.float32) + gb2_ref[...])
    ragl = jnp.dot(rag_ref[...], rfWT_ref[...], preferred_element_type=jnp.float32) + rfb_ref[...]
    out = jnp.dot(h, foWT_ref[...], preferred_element_type=jnp.float32) + fob_ref[...]
    out_ref[...] = out + gate * ragl


def _forecast(xT, q, rag_flat, WihT, WhhT, bih, bhh, foWT, fob, rfWT, rfb,
              gW1T, gb1, gW2T, gb2):
    return pl.pallas_call(
        _forecast_kernel,
        out_shape=jax.ShapeDtypeStruct((B, FS * OD), jnp.float32),
    )(xT, q, rag_flat, WihT, WhhT, bih, bhh, foWT, fob, rfWT, rfb,
      gW1T, gb1, gW2T, gb2)


@jax.jit
def kernel(x, W_ih, W_hh, b_ih, b_hh, fo_W, fo_b, rf_W, rf_b,
           g_W1, g_b1, g_W2, g_b2, meta_sequences, meta_labels):
    q = x[:, :, 3]
    metaT = jnp.pad(meta_sequences.T, ((0, 0), (0, M_PAD - M)))
    idx8 = _topk(q, metaT)
    idx = idx8[:, :TK]

    labels_pad = jnp.pad(meta_labels, ((0, 0), (0, _DPAD - FS)))
    rows = _sc_gather(labels_pad, idx.reshape(_BTK))
    rag_flat = rows[:, :FS].reshape(B, TK * FS)

    xT = jnp.transpose(x, (1, 0, 2))
    out = _forecast(
        xT, q, rag_flat,
        W_ih.T, W_hh.T, b_ih[None, :], b_hh[None, :],
        fo_W.T, fo_b[None, :], rf_W.T, rf_b[None, :],
        g_W1.T, g_b1[None, :], g_W2.T, g_b2[None, :],
    )
    return out


# speedup vs baseline: 1.9862x; 1.9862x over previous
import functools

import jax
import jax.numpy as jnp
from jax import lax
from jax.experimental import pallas as pl
from jax.experimental.pallas import tpu as pltpu
from jax.experimental.pallas import tpu_sc as plsc

B, T, F, H, FS, OD, TK = 1024, 50, 8, 64, 8, 1, 4
M = 100000
MT = 2048
M_PAD = ((M + MT - 1) // MT) * MT
N_TILES = M_PAD // MT
BIG_I = 2**30
INF = float("inf")


INT_MAX = 2**31 - 1
NCOL = MT // 128


def _cand_kernel(q_ref, metaT_ref, keys_ref, idx_ref):
    pid = pl.program_id(0)
    q = q_ref[...]
    mt = metaT_ref[...]
    norms = jnp.sum(mt * mt, axis=0, keepdims=True)
    qn = jnp.sum(q * q, axis=1, keepdims=True)
    qm = jnp.dot(q, mt, preferred_element_type=jnp.float32)
    d2 = (norms - 2.0 * qm) + qn

    lane = lax.broadcasted_iota(jnp.int32, (1, MT), 1)
    grp = lane >> 7
    key = (lax.bitcast_convert_type(d2, jnp.int32) & -16) | grp

    nodes = [(key[:, c * 128:(c + 1) * 128],) for c in range(NCOL)]

    def _comb(a, b):
        lo = jnp.minimum(a[0], b[0])
        hi = jnp.maximum(a[0], b[0])
        if len(a) == 1:
            return (lo, hi)
        return (lo, jnp.minimum(hi, jnp.minimum(a[1], b[1])))

    while len(nodes) > 1:
        nodes = [_comb(nodes[i], nodes[i + 1]) for i in range(0, len(nodes), 2)]
    fk = jnp.concatenate([nodes[0][0], nodes[0][1]], axis=1)

    lane256 = lax.broadcasted_iota(jnp.int32, (1, 256), 1)
    off = pid * MT
    ks, gs = [], []
    for _ in range(TK):
        m = jnp.min(fk, axis=1, keepdims=True)
        am = jnp.min(jnp.where(fk == m, lane256, BIG_I), axis=1, keepdims=True)
        ks.append(m)
        gs.append(off + ((m & 15) << 7) + (am & 127))
        fk = jnp.where(lane256 == am, INT_MAX, fk)
    keys_ref[...] = jnp.concatenate(ks, axis=1)[None]
    idx_ref[...] = jnp.concatenate(gs, axis=1)[None]


def _merge_kernel(keys_ref, cidx_ref, idx_ref):
    keys = keys_ref[...]
    cidx = cidx_ref[...]
    n = N_TILES * TK
    lane = lax.broadcasted_iota(jnp.int32, (1, n), 1)
    outs = []
    for _ in range(TK):
        m = jnp.min(keys, axis=1, keepdims=True)
        am = jnp.min(jnp.where(keys == m, lane, BIG_I), axis=1, keepdims=True)
        outs.append(jnp.min(jnp.where(lane == am, cidx, BIG_I), axis=1, keepdims=True))
        keys = jnp.where(lane == am, INT_MAX, keys)
    idx_ref[...] = jnp.concatenate(outs + [jnp.full((B, 1), BIG_I, jnp.int32)] * (8 - TK), axis=1)


def _topk(q, metaT_pad):
    keys, cidx = pl.pallas_call(
        _cand_kernel,
        grid=(N_TILES,),
        in_specs=[
            pl.BlockSpec((B, T), lambda i: (0, 0)),
            pl.BlockSpec((T, MT), lambda i: (0, i)),
        ],
        out_specs=[
            pl.BlockSpec((1, B, TK), lambda i: (i, 0, 0)),
            pl.BlockSpec((1, B, TK), lambda i: (i, 0, 0)),
        ],
        out_shape=[
            jax.ShapeDtypeStruct((N_TILES, B, TK), jnp.int32),
            jax.ShapeDtypeStruct((N_TILES, B, TK), jnp.int32),
        ],
    )(q, metaT_pad)
    keys = keys.transpose(1, 0, 2).reshape(B, N_TILES * TK)
    cidx = cidx.transpose(1, 0, 2).reshape(B, N_TILES * TK)
    return pl.pallas_call(
        _merge_kernel,
        out_shape=jax.ShapeDtypeStruct((B, 8), jnp.int32),
    )(keys, cidx)


_NC, _NS = 2, 16
_NW = _NC * _NS
_BTK = B * TK
_BPW = _BTK // _NW
_DPAD = 128


def _sc_gather(table_pad, idx_flat):
    mesh = plsc.VectorSubcoreMesh(core_axis_name="c", subcore_axis_name="s")

    @functools.partial(
        pl.kernel, mesh=mesh,
        out_type=jax.ShapeDtypeStruct((_BTK, _DPAD), jnp.float32),
        scratch_types=[
            pltpu.VMEM((_BPW,), jnp.int32),
            pltpu.VMEM((_BPW, _DPAD), jnp.float32),
            pltpu.SemaphoreType.DMA,
        ],
    )
    def k(table_hbm, idx_hbm, out_hbm, idx_v, rows_v, sem):
        wid = lax.axis_index("s") * _NC + lax.axis_index("c")
        base = wid * _BPW
        pltpu.sync_copy(idx_hbm.at[pl.ds(base, _BPW)], idx_v)
        pltpu.async_copy(table_hbm.at[idx_v], rows_v, sem).wait()
        pltpu.sync_copy(rows_v, out_hbm.at[pl.ds(base, _BPW)])

    return k(table_pad, idx_flat)


def _forecast_kernel(xT_ref, q_ref, rag_ref,
                     WihT_ref, WhhT_ref, bih_ref, bhh_ref,
                     foWT_ref, fob_ref, rfWT_ref, rfb_ref,
                     gW1T_ref, gb1_ref, gW2T_ref, gb2_ref,
                     out_ref):
    WihT = WihT_ref[...]
    WhhT = WhhT_ref[...]
    bih = bih_ref[...]
    bhh = bhh_ref[...]

    def step(t, h):
        xt = xT_ref[t]
        gi = jnp.dot(xt, WihT, preferred_element_type=jnp.float32) + bih
        gh = jnp.dot(h, WhhT, preferred_element_type=jnp.float32) + bhh
        r = jax.nn.sigmoid(gi[:, :H] + gh[:, :H])
        z = jax.nn.sigmoid(gi[:, H:2 * H] + gh[:, H:2 * H])
        n = jnp.tanh(gi[:, 2 * H:] + r * gh[:, 2 * H:])
        return (1.0 - z) * n + z * h

    h = lax.fori_loop(0, T, step, jnp.zeros((B, H), jnp.float32))

    q = q_ref[...]
    t1 = jnp.tanh(jnp.dot(q, gW1T_ref[...], preferred_element_type=jnp.float32) + gb1_ref[...])
    gate = jax.nn.sigmoid(jnp.dot(t1, gW2T_ref[...], preferred_element_type=jnp.float32) + gb2_ref[...])
    ragl = jnp.dot(rag_ref[...], rfWT_ref[...], preferred_element_type=jnp.float32) + rfb_ref[...]
    out = jnp.dot(h, foWT_ref[...], preferred_element_type=jnp.float32) + fob_ref[...]
    out_ref[...] = out + gate * ragl


def _forecast(xT, q, rag_flat, WihT, WhhT, bih, bhh, foWT, fob, rfWT, rfb,
              gW1T, gb1, gW2T, gb2):
    return pl.pallas_call(
        _forecast_kernel,
        out_shape=jax.ShapeDtypeStruct((B, FS * OD), jnp.float32),
    )(xT, q, rag_flat, WihT, WhhT, bih, bhh, foWT, fob, rfWT, rfb,
      gW1T, gb1, gW2T, gb2)


@jax.jit
def kernel(x, W_ih, W_hh, b_ih, b_hh, fo_W, fo_b, rf_W, rf_b,
           g_W1, g_b1, g_W2, g_b2, meta_sequences, meta_labels):
    q = x[:, :, 3]
    metaT = jnp.pad(meta_sequences.T, ((0, 0), (0, M_PAD - M)),
                    constant_values=1e18)
    idx8 = _topk(q, metaT)
    idx = idx8[:, :TK]

    labels_pad = jnp.pad(meta_labels, ((0, 0), (0, _DPAD - FS)))
    rows = _sc_gather(labels_pad, idx.reshape(_BTK))
    rag_flat = rows[:, :FS].reshape(B, TK * FS)

    xT = jnp.transpose(x, (1, 0, 2))
    out = _forecast(
        xT, q, rag_flat,
        W_ih.T, W_hh.T, b_ih[None, :], b_hh[None, :],
        fo_W.T, fo_b[None, :], rf_W.T, rf_b[None, :],
        g_W1.T, g_b1[None, :], g_W2.T, g_b2[None, :],
    )
    return out


# speedup vs baseline: 2.0862x; 1.0503x over previous
import functools

import jax
import jax.numpy as jnp
from jax import lax
from jax.experimental import pallas as pl
from jax.experimental.pallas import tpu as pltpu
from jax.experimental.pallas import tpu_sc as plsc

B, T, F, H, FS, OD, TK = 1024, 50, 8, 64, 8, 1, 4
M = 100000
MT = 2048
M_PAD = ((M + MT - 1) // MT) * MT
N_TILES = M_PAD // MT
BIG_I = 2**30
INF = float("inf")


INT_MAX = 2**31 - 1
NCOL = MT // 128


def _cand_kernel(q_ref, metaT_ref, keys_ref, idx_ref):
    pid = pl.program_id(0)
    q = q_ref[...]
    mt = metaT_ref[...]
    norms = jnp.sum(mt * mt, axis=0, keepdims=True)
    qm = jnp.dot(q, mt, preferred_element_type=jnp.float32)
    s = norms - 2.0 * qm

    lane128 = lax.broadcasted_iota(jnp.int32, (1, 128), 1)
    nodes = [(s[:, c * 128:(c + 1) * 128], lane128 + c * 128)
             for c in range(NCOL)]

    def _comb(a, b):
        if len(a) == 2:
            av, ap = a
            bv, bp = b
            t = av <= bv
            return (jnp.minimum(av, bv), jnp.where(t, ap, bp),
                    jnp.maximum(av, bv), jnp.where(t, bp, ap))
        av, ap, av2, ap2 = a
        bv, bp, bv2, bp2 = b
        t = av <= bv
        m1v = jnp.minimum(av, bv)
        m1p = jnp.where(t, ap, bp)
        lv = jnp.maximum(av, bv)
        lp = jnp.where(t, bp, ap)
        t2 = av2 <= bv2
        cv = jnp.minimum(av2, bv2)
        cp = jnp.where(t2, ap2, bp2)
        t3 = cv <= lv
        return (m1v, m1p, jnp.minimum(cv, lv), jnp.where(t3, cp, lp))

    while len(nodes) > 1:
        nodes = [_comb(nodes[i], nodes[i + 1]) for i in range(0, len(nodes), 2)]
    fv = jnp.concatenate([nodes[0][0], nodes[0][2]], axis=1)
    fp = jnp.concatenate([nodes[0][1], nodes[0][3]], axis=1)

    lane256 = lax.broadcasted_iota(jnp.int32, (1, 256), 1)
    off = pid * MT
    ks, gs = [], []
    for _ in range(TK):
        m = jnp.min(fv, axis=1, keepdims=True)
        mp = jnp.min(jnp.where(fv == m, fp, BIG_I), axis=1, keepdims=True)
        slot = jnp.min(jnp.where((fv == m) & (fp == mp), lane256, BIG_I),
                       axis=1, keepdims=True)
        ks.append(m)
        gs.append(off + mp)
        fv = jnp.where(lane256 == slot, INF, fv)
    keys_ref[...] = jnp.concatenate(ks, axis=1)[None]
    idx_ref[...] = jnp.concatenate(gs, axis=1)[None]


def _merge_kernel(keys_ref, cidx_ref, idx_ref):
    keys = keys_ref[...]
    cidx = cidx_ref[...]
    n = N_TILES * TK
    lane = lax.broadcasted_iota(jnp.int32, (1, n), 1)
    outs = []
    for _ in range(TK):
        m = jnp.min(keys, axis=1, keepdims=True)
        mi = jnp.min(jnp.where(keys == m, cidx, BIG_I), axis=1, keepdims=True)
        slot = jnp.min(jnp.where((keys == m) & (cidx == mi), lane, BIG_I),
                       axis=1, keepdims=True)
        outs.append(mi)
        keys = jnp.where(lane == slot, INF, keys)
    idx_ref[...] = jnp.concatenate(
        outs + [jnp.full((B, 1), BIG_I, jnp.int32)] * (8 - TK), axis=1)


def _topk(q, metaT_pad):
    keys, cidx = pl.pallas_call(
        _cand_kernel,
        grid=(N_TILES,),
        in_specs=[
            pl.BlockSpec((B, T), lambda i: (0, 0)),
            pl.BlockSpec((T, MT), lambda i: (0, i)),
        ],
        out_specs=[
            pl.BlockSpec((1, B, TK), lambda i: (i, 0, 0)),
            pl.BlockSpec((1, B, TK), lambda i: (i, 0, 0)),
        ],
        out_shape=[
            jax.ShapeDtypeStruct((N_TILES, B, TK), jnp.float32),
            jax.ShapeDtypeStruct((N_TILES, B, TK), jnp.int32),
        ],
    )(q, metaT_pad)
    keys = keys.transpose(1, 0, 2).reshape(B, N_TILES * TK)
    cidx = cidx.transpose(1, 0, 2).reshape(B, N_TILES * TK)
    return pl.pallas_call(
        _merge_kernel,
        out_shape=jax.ShapeDtypeStruct((B, 8), jnp.int32),
    )(keys, cidx)


_NC, _NS = 2, 16
_NW = _NC * _NS
_BTK = B * TK
_BPW = _BTK // _NW
_DPAD = 128


def _sc_gather(table_pad, idx_flat):
    mesh = plsc.VectorSubcoreMesh(core_axis_name="c", subcore_axis_name="s")

    nchunk = _BPW // 128

    @functools.partial(
        pl.kernel, mesh=mesh,
        out_type=jax.ShapeDtypeStruct((_BTK, _DPAD), jnp.float32),
        scratch_types=[
            pltpu.VMEM((nchunk, 128), jnp.int32),
            pltpu.VMEM((_BPW, _DPAD), jnp.float32),
            pltpu.SemaphoreType.DMA,
        ],
    )
    def k(table_hbm, idx_hbm, out_hbm, idx_v, rows_v, sem):
        wid = lax.axis_index("s") * _NC + lax.axis_index("c")
        base = wid * _BPW
        for j in range(nchunk):
            pltpu.sync_copy(idx_hbm.at[pl.ds(base + j * 128, 128)], idx_v.at[j])
        copies = [
            pltpu.async_copy(table_hbm.at[idx_v.at[j]],
                             rows_v.at[pl.ds(j * 128, 128)], sem)
            for j in range(nchunk)
        ]
        for c in copies:
            c.wait()
        pltpu.sync_copy(rows_v, out_hbm.at[pl.ds(base, _BPW)])

    return k(table_pad, idx_flat)


def _forecast_kernel(xT_ref, q_ref, rag_ref,
                     WihT_ref, WhhT_ref, bih_ref, bhh_ref,
                     foWT_ref, fob_ref, rfWT_ref, rfb_ref,
                     gW1T_ref, gb1_ref, gW2T_ref, gb2_ref,
                     out_ref):
    WihT = WihT_ref[...]
    WhhT = WhhT_ref[...]
    bih = bih_ref[...]
    bhh = bhh_ref[...]

    def step(t, h):
        xt = xT_ref[t]
        gi = jnp.dot(xt, WihT, preferred_element_type=jnp.float32) + bih
        gh = jnp.dot(h, WhhT, preferred_element_type=jnp.float32) + bhh
        r = jax.nn.sigmoid(gi[:, :H] + gh[:, :H])
        z = jax.nn.sigmoid(gi[:, H:2 * H] + gh[:, H:2 * H])
        n = jnp.tanh(gi[:, 2 * H:] + r * gh[:, 2 * H:])
        return (1.0 - z) * n + z * h

    h = lax.fori_loop(0, T, step, jnp.zeros((B, H), jnp.float32))

    q = q_ref[...]
    rag = rag_ref[...]
    t1 = jnp.tanh(jnp.dot(q, gW1T_ref[...], preferred_element_type=jnp.float32) + gb1_ref[...])
    gate = jax.nn.sigmoid(jnp.dot(t1, gW2T_ref[...], preferred_element_type=jnp.float32) + gb2_ref[...])
    ragl = jnp.dot(rag, rfWT_ref[...], preferred_element_type=jnp.float32) + rfb_ref[...]
    out = jnp.dot(h, foWT_ref[...], preferred_element_type=jnp.float32) + fob_ref[...]
    out_ref[...] = out + gate * ragl


def _forecast(xT, q, rag, WihT, WhhT, bih, bhh, foWT, fob, rfWT, rfb,
              gW1T, gb1, gW2T, gb2):
    return pl.pallas_call(
        _forecast_kernel,
        out_shape=jax.ShapeDtypeStruct((B, FS * OD), jnp.float32),
    )(xT, q, rag, WihT, WhhT, bih, bhh, foWT, fob, rfWT, rfb,
      gW1T, gb1, gW2T, gb2)


@jax.jit
def kernel(x, W_ih, W_hh, b_ih, b_hh, fo_W, fo_b, rf_W, rf_b,
           g_W1, g_b1, g_W2, g_b2, meta_sequences, meta_labels):
    q = x[:, :, 3]
    metaT = jnp.pad(meta_sequences.T, ((0, 0), (0, M_PAD - M)),
                    constant_values=1e18)
    idx8 = _topk(q, metaT)
    idx = idx8[:, :TK]

    labels_pad = jnp.pad(meta_labels, ((0, 0), (0, _DPAD - FS)))
    rows = _sc_gather(labels_pad, idx.reshape(_BTK))
    rag = rows[:, :FS].reshape(B, TK * FS)

    xT = jnp.transpose(x, (1, 0, 2))
    out = _forecast(
        xT, q, rag,
        W_ih.T, W_hh.T, b_ih[None, :], b_hh[None, :],
        fo_W.T, fo_b[None, :], rf_W.T, rf_b[None, :],
        g_W1.T, g_b1[None, :], g_W2.T, g_b2[None, :],
    )
    return out


# speedup vs baseline: 2.4497x; 1.1742x over previous
import functools

import jax
import jax.numpy as jnp
from jax import lax
from jax.experimental import pallas as pl
from jax.experimental.pallas import tpu as pltpu
from jax.experimental.pallas import tpu_sc as plsc

B, T, F, H, FS, OD, TK = 1024, 50, 8, 64, 8, 1, 4
M = 100000
MT = 4096
M_PAD = ((M + MT - 1) // MT) * MT
N_TILES = M_PAD // MT
BIG_I = 2**30
INF = float("inf")


INT_MAX = 2**31 - 1
NCOL = MT // 128


def _cand_kernel(q_ref, metaT_ref, keys_ref, idx_ref):
    pid = pl.program_id(0)
    q = q_ref[...]
    mt = metaT_ref[...]
    norms = jnp.sum(mt * mt, axis=0, keepdims=True)
    qm = jnp.dot(q, mt, preferred_element_type=jnp.float32)
    s = norms - 2.0 * qm

    lane128 = lax.broadcasted_iota(jnp.int32, (1, 128), 1)
    nodes = [(s[:, c * 128:(c + 1) * 128], lane128 + c * 128)
             for c in range(NCOL)]

    def _comb(a, b):
        if len(a) == 2:
            av, ap = a
            bv, bp = b
            t = av <= bv
            return (jnp.minimum(av, bv), jnp.where(t, ap, bp),
                    jnp.maximum(av, bv), jnp.where(t, bp, ap))
        av, ap, av2, ap2 = a
        bv, bp, bv2, bp2 = b
        t = av <= bv
        m1v = jnp.minimum(av, bv)
        m1p = jnp.where(t, ap, bp)
        lv = jnp.maximum(av, bv)
        lp = jnp.where(t, bp, ap)
        t2 = av2 <= bv2
        cv = jnp.minimum(av2, bv2)
        cp = jnp.where(t2, ap2, bp2)
        t3 = cv <= lv
        return (m1v, m1p, jnp.minimum(cv, lv), jnp.where(t3, cp, lp))

    while len(nodes) > 1:
        nodes = [_comb(nodes[i], nodes[i + 1]) for i in range(0, len(nodes), 2)]
    fv = jnp.concatenate([nodes[0][0], nodes[0][2]], axis=1)
    fp = jnp.concatenate([nodes[0][1], nodes[0][3]], axis=1)

    lane256 = lax.broadcasted_iota(jnp.int32, (1, 256), 1)
    off = pid * MT
    ks, gs = [], []
    for _ in range(TK):
        m = jnp.min(fv, axis=1, keepdims=True)
        mp = jnp.min(jnp.where(fv == m, fp, BIG_I), axis=1, keepdims=True)
        slot = jnp.min(jnp.where((fv == m) & (fp == mp), lane256, BIG_I),
                       axis=1, keepdims=True)
        ks.append(m)
        gs.append(off + mp)
        fv = jnp.where(lane256 == slot, INF, fv)
    keys_ref[...] = jnp.concatenate(ks, axis=1)[None]
    idx_ref[...] = jnp.concatenate(gs, axis=1)[None]


def _merge_kernel(keys_ref, cidx_ref, idx_ref):
    keys = keys_ref[...]
    cidx = cidx_ref[...]
    n = N_TILES * TK
    lane = lax.broadcasted_iota(jnp.int32, (1, n), 1)
    outs = []
    for _ in range(TK):
        m = jnp.min(keys, axis=1, keepdims=True)
        mi = jnp.min(jnp.where(keys == m, cidx, BIG_I), axis=1, keepdims=True)
        slot = jnp.min(jnp.where((keys == m) & (cidx == mi), lane, BIG_I),
                       axis=1, keepdims=True)
        outs.append(mi)
        keys = jnp.where(lane == slot, INF, keys)
    idx_ref[...] = jnp.concatenate(
        outs + [jnp.full((B, 1), BIG_I, jnp.int32)] * (8 - TK), axis=1)


def _topk(q, metaT_pad):
    keys, cidx = pl.pallas_call(
        _cand_kernel,
        grid=(N_TILES,),
        in_specs=[
            pl.BlockSpec((B, T), lambda i: (0, 0)),
            pl.BlockSpec((T, MT), lambda i: (0, i)),
        ],
        out_specs=[
            pl.BlockSpec((1, B, TK), lambda i: (i, 0, 0)),
            pl.BlockSpec((1, B, TK), lambda i: (i, 0, 0)),
        ],
        out_shape=[
            jax.ShapeDtypeStruct((N_TILES, B, TK), jnp.float32),
            jax.ShapeDtypeStruct((N_TILES, B, TK), jnp.int32),
        ],
    )(q, metaT_pad)
    keys = keys.transpose(1, 0, 2).reshape(B, N_TILES * TK)
    cidx = cidx.transpose(1, 0, 2).reshape(B, N_TILES * TK)
    return pl.pallas_call(
        _merge_kernel,
        out_shape=jax.ShapeDtypeStruct((B, 8), jnp.int32),
    )(keys, cidx)


_NC, _NS = 2, 16
_NW = _NC * _NS
_BTK = B * TK
_BPW = _BTK // _NW
_DPAD = 128


def _sc_gather(table_pad, idx_flat):
    mesh = plsc.VectorSubcoreMesh(core_axis_name="c", subcore_axis_name="s")

    nchunk = _BPW // 128

    @functools.partial(
        pl.kernel, mesh=mesh,
        out_type=jax.ShapeDtypeStruct((_BTK, _DPAD), jnp.float32),
        scratch_types=[
            pltpu.VMEM((nchunk, 128), jnp.int32),
            pltpu.VMEM((_BPW, _DPAD), jnp.float32),
            pltpu.SemaphoreType.DMA,
        ],
    )
    def k(table_hbm, idx_hbm, out_hbm, idx_v, rows_v, sem):
        wid = lax.axis_index("s") * _NC + lax.axis_index("c")
        base = wid * _BPW
        for j in range(nchunk):
            pltpu.sync_copy(idx_hbm.at[pl.ds(base + j * 128, 128)], idx_v.at[j])
        copies = [
            pltpu.async_copy(table_hbm.at[idx_v.at[j]],
                             rows_v.at[pl.ds(j * 128, 128)], sem)
            for j in range(nchunk)
        ]
        for c in copies:
            c.wait()
        pltpu.sync_copy(rows_v, out_hbm.at[pl.ds(base, _BPW)])

    return k(table_pad, idx_flat)


def _forecast_kernel(xT_ref, q_ref, rag_ref,
                     WihT_ref, WhhT_ref, bih_ref, bhh_ref,
                     foWT_ref, fob_ref, rfWT_ref, rfb_ref,
                     gW1T_ref, gb1_ref, gW2T_ref, gb2_ref,
                     out_ref):
    WihT = WihT_ref[...]
    WhhT = WhhT_ref[...]
    bih = bih_ref[...]
    bhh = bhh_ref[...]

    def step(t, h):
        xt = xT_ref[t]
        gi = jnp.dot(xt, WihT, preferred_element_type=jnp.float32) + bih
        gh = jnp.dot(h, WhhT, preferred_element_type=jnp.float32) + bhh
        r = jax.nn.sigmoid(gi[:, :H] + gh[:, :H])
        z = jax.nn.sigmoid(gi[:, H:2 * H] + gh[:, H:2 * H])
        n = jnp.tanh(gi[:, 2 * H:] + r * gh[:, 2 * H:])
        return (1.0 - z) * n + z * h

    h = lax.fori_loop(0, T, step, jnp.zeros((B, H), jnp.float32))

    q = q_ref[...]
    rag = rag_ref[...]
    t1 = jnp.tanh(jnp.dot(q, gW1T_ref[...], preferred_element_type=jnp.float32) + gb1_ref[...])
    gate = jax.nn.sigmoid(jnp.dot(t1, gW2T_ref[...], preferred_element_type=jnp.float32) + gb2_ref[...])
    ragl = jnp.dot(rag, rfWT_ref[...], preferred_element_type=jnp.float32) + rfb_ref[...]
    out = jnp.dot(h, foWT_ref[...], preferred_element_type=jnp.float32) + fob_ref[...]
    out_ref[...] = out + gate * ragl


def _forecast(xT, q, rag, WihT, WhhT, bih, bhh, foWT, fob, rfWT, rfb,
              gW1T, gb1, gW2T, gb2):
    return pl.pallas_call(
        _forecast_kernel,
        out_shape=jax.ShapeDtypeStruct((B, FS * OD), jnp.float32),
    )(xT, q, rag, WihT, WhhT, bih, bhh, foWT, fob, rfWT, rfb,
      gW1T, gb1, gW2T, gb2)


@jax.jit
def kernel(x, W_ih, W_hh, b_ih, b_hh, fo_W, fo_b, rf_W, rf_b,
           g_W1, g_b1, g_W2, g_b2, meta_sequences, meta_labels):
    q = x[:, :, 3]
    metaT = jnp.pad(meta_sequences.T, ((0, 0), (0, M_PAD - M)),
                    constant_values=1e18)
    idx8 = _topk(q, metaT)
    idx = idx8[:, :TK]

    labels_pad = jnp.pad(meta_labels, ((0, 0), (0, _DPAD - FS)))
    rows = _sc_gather(labels_pad, idx.reshape(_BTK))
    rag = rows[:, :FS].reshape(B, TK * FS)

    xT = jnp.transpose(x, (1, 0, 2))
    out = _forecast(
        xT, q, rag,
        W_ih.T, W_hh.T, b_ih[None, :], b_hh[None, :],
        fo_W.T, fo_b[None, :], rf_W.T, rf_b[None, :],
        g_W1.T, g_b1[None, :], g_W2.T, g_b2[None, :],
    )
    return out


# speedup vs baseline: 2.6514x; 1.0824x over previous
import functools

import jax
import jax.numpy as jnp
from jax import lax
from jax.experimental import pallas as pl
from jax.experimental.pallas import tpu as pltpu
from jax.experimental.pallas import tpu_sc as plsc

B, T, F, H, FS, OD, TK = 1024, 50, 8, 64, 8, 1, 4
M = 100000
MT = 8192
M_PAD = ((M + MT - 1) // MT) * MT
N_TILES = M_PAD // MT
BIG_I = 2**30
INF = float("inf")


INT_MAX = 2**31 - 1
NCOL = MT // 128


def _cand_kernel(q_ref, metaT_ref, keys_ref, idx_ref):
    pid = pl.program_id(0)
    q = q_ref[...]
    mt = metaT_ref[...]
    norms = jnp.sum(mt * mt, axis=0, keepdims=True)
    qm = jnp.dot(q, mt, preferred_element_type=jnp.float32)
    s = norms - 2.0 * qm

    lane128 = lax.broadcasted_iota(jnp.int32, (1, 128), 1)
    nodes = [(s[:, c * 128:(c + 1) * 128], lane128 + c * 128)
             for c in range(NCOL)]

    def _comb(a, b):
        if len(a) == 2:
            av, ap = a
            bv, bp = b
            t = av <= bv
            return (jnp.minimum(av, bv), jnp.where(t, ap, bp),
                    jnp.maximum(av, bv), jnp.where(t, bp, ap))
        av, ap, av2, ap2 = a
        bv, bp, bv2, bp2 = b
        t = av <= bv
        m1v = jnp.minimum(av, bv)
        m1p = jnp.where(t, ap, bp)
        lv = jnp.maximum(av, bv)
        lp = jnp.where(t, bp, ap)
        t2 = av2 <= bv2
        cv = jnp.minimum(av2, bv2)
        cp = jnp.where(t2, ap2, bp2)
        t3 = cv <= lv
        return (m1v, m1p, jnp.minimum(cv, lv), jnp.where(t3, cp, lp))

    while len(nodes) > 1:
        nodes = [_comb(nodes[i], nodes[i + 1]) for i in range(0, len(nodes), 2)]
    fv = jnp.concatenate([nodes[0][0], nodes[0][2]], axis=1)
    fp = jnp.concatenate([nodes[0][1], nodes[0][3]], axis=1)

    lane256 = lax.broadcasted_iota(jnp.int32, (1, 256), 1)
    off = pid * MT
    ks, gs = [], []
    for _ in range(TK):
        m = jnp.min(fv, axis=1, keepdims=True)
        mp = jnp.min(jnp.where(fv == m, fp, BIG_I), axis=1, keepdims=True)
        slot = jnp.min(jnp.where((fv == m) & (fp == mp), lane256, BIG_I),
                       axis=1, keepdims=True)
        ks.append(m)
        gs.append(off + mp)
        fv = jnp.where(lane256 == slot, INF, fv)
    keys_ref[...] = jnp.concatenate(ks, axis=1)[None]
    idx_ref[...] = jnp.concatenate(gs, axis=1)[None]


def _merge_kernel(keys_ref, cidx_ref, idx_ref):
    keys = keys_ref[...]
    cidx = cidx_ref[...]
    n = N_TILES * TK
    lane = lax.broadcasted_iota(jnp.int32, (1, n), 1)
    outs = []
    for _ in range(TK):
        m = jnp.min(keys, axis=1, keepdims=True)
        mi = jnp.min(jnp.where(keys == m, cidx, BIG_I), axis=1, keepdims=True)
        slot = jnp.min(jnp.where((keys == m) & (cidx == mi), lane, BIG_I),
                       axis=1, keepdims=True)
        outs.append(mi)
        keys = jnp.where(lane == slot, INF, keys)
    idx_ref[...] = jnp.concatenate(
        outs + [jnp.full((B, 1), BIG_I, jnp.int32)] * (8 - TK), axis=1)


def _topk(q, metaT_pad):
    keys, cidx = pl.pallas_call(
        _cand_kernel,
        grid=(N_TILES,),
        in_specs=[
            pl.BlockSpec((B, T), lambda i: (0, 0)),
            pl.BlockSpec((T, MT), lambda i: (0, i)),
        ],
        out_specs=[
            pl.BlockSpec((1, B, TK), lambda i: (i, 0, 0)),
            pl.BlockSpec((1, B, TK), lambda i: (i, 0, 0)),
        ],
        out_shape=[
            jax.ShapeDtypeStruct((N_TILES, B, TK), jnp.float32),
            jax.ShapeDtypeStruct((N_TILES, B, TK), jnp.int32),
        ],
    )(q, metaT_pad)
    keys = keys.transpose(1, 0, 2).reshape(B, N_TILES * TK)
    cidx = cidx.transpose(1, 0, 2).reshape(B, N_TILES * TK)
    return pl.pallas_call(
        _merge_kernel,
        out_shape=jax.ShapeDtypeStruct((B, 8), jnp.int32),
    )(keys, cidx)


_NC, _NS = 2, 16
_NW = _NC * _NS
_BTK = B * TK
_BPW = _BTK // _NW
_DPAD = 128


def _sc_gather(table_pad, idx_flat):
    mesh = plsc.VectorSubcoreMesh(core_axis_name="c", subcore_axis_name="s")

    nchunk = _BPW // 128

    @functools.partial(
        pl.kernel, mesh=mesh,
        out_type=jax.ShapeDtypeStruct((_BTK, _DPAD), jnp.float32),
        scratch_types=[
            pltpu.VMEM((nchunk, 128), jnp.int32),
            pltpu.VMEM((_BPW, _DPAD), jnp.float32),
            pltpu.SemaphoreType.DMA,
        ],
    )
    def k(table_hbm, idx_hbm, out_hbm, idx_v, rows_v, sem):
        wid = lax.axis_index("s") * _NC + lax.axis_index("c")
        base = wid * _BPW
        for j in range(nchunk):
            pltpu.sync_copy(idx_hbm.at[pl.ds(base + j * 128, 128)], idx_v.at[j])
        copies = [
            pltpu.async_copy(table_hbm.at[idx_v.at[j]],
                             rows_v.at[pl.ds(j * 128, 128)], sem)
            for j in range(nchunk)
        ]
        for c in copies:
            c.wait()
        pltpu.sync_copy(rows_v, out_hbm.at[pl.ds(base, _BPW)])

    return k(table_pad, idx_flat)


def _forecast_kernel(xT_ref, q_ref, rag_ref,
                     WihT_ref, WhhT_ref, bih_ref, bhh_ref,
                     foWT_ref, fob_ref, rfWT_ref, rfb_ref,
                     gW1T_ref, gb1_ref, gW2T_ref, gb2_ref,
                     out_ref):
    WihT = WihT_ref[...]
    WhhT = WhhT_ref[...]
    bih = bih_ref[...]
    bhh = bhh_ref[...]

    def step(t, h):
        xt = xT_ref[t]
        gi = jnp.dot(xt, WihT, preferred_element_type=jnp.float32) + bih
        gh = jnp.dot(h, WhhT, preferred_element_type=jnp.float32) + bhh
        r = jax.nn.sigmoid(gi[:, :H] + gh[:, :H])
        z = jax.nn.sigmoid(gi[:, H:2 * H] + gh[:, H:2 * H])
        n = jnp.tanh(gi[:, 2 * H:] + r * gh[:, 2 * H:])
        return (1.0 - z) * n + z * h

    h = lax.fori_loop(0, T, step, jnp.zeros((B, H), jnp.float32))

    q = q_ref[...]
    rag = rag_ref[...]
    t1 = jnp.tanh(jnp.dot(q, gW1T_ref[...], preferred_element_type=jnp.float32) + gb1_ref[...])
    gate = jax.nn.sigmoid(jnp.dot(t1, gW2T_ref[...], preferred_element_type=jnp.float32) + gb2_ref[...])
    ragl = jnp.dot(rag, rfWT_ref[...], preferred_element_type=jnp.float32) + rfb_ref[...]
    out = jnp.dot(h, foWT_ref[...], preferred_element_type=jnp.float32) + fob_ref[...]
    out_ref[...] = out + gate * ragl


def _forecast(xT, q, rag, WihT, WhhT, bih, bhh, foWT, fob, rfWT, rfb,
              gW1T, gb1, gW2T, gb2):
    return pl.pallas_call(
        _forecast_kernel,
        out_shape=jax.ShapeDtypeStruct((B, FS * OD), jnp.float32),
    )(xT, q, rag, WihT, WhhT, bih, bhh, foWT, fob, rfWT, rfb,
      gW1T, gb1, gW2T, gb2)


@jax.jit
def kernel(x, W_ih, W_hh, b_ih, b_hh, fo_W, fo_b, rf_W, rf_b,
           g_W1, g_b1, g_W2, g_b2, meta_sequences, meta_labels):
    q = x[:, :, 3]
    metaT = jnp.pad(meta_sequences.T, ((0, 0), (0, M_PAD - M)),
                    constant_values=1e18)
    idx8 = _topk(q, metaT)
    idx = idx8[:, :TK]

    labels_pad = jnp.pad(meta_labels, ((0, 0), (0, _DPAD - FS)))
    rows = _sc_gather(labels_pad, idx.reshape(_BTK))
    rag = rows[:, :FS].reshape(B, TK * FS)

    xT = jnp.transpose(x, (1, 0, 2))
    out = _forecast(
        xT, q, rag,
        W_ih.T, W_hh.T, b_ih[None, :], b_hh[None, :],
        fo_W.T, fo_b[None, :], rf_W.T, rf_b[None, :],
        g_W1.T, g_b1[None, :], g_W2.T, g_b2[None, :],
    )
    return out
